# baseline (speedup 1.0000x reference)
import functools

import jax
import jax.numpy as jnp
from jax import lax
from jax.experimental import pallas as pl
from jax.experimental.pallas import tpu as pltpu
from jax.experimental.pallas import tpu_sc as plsc

N_NODES = 100000
N_EDGES = 3200000
NF = 6
NC, NS = 2, 16
NW = NC * NS
EPW = N_EDGES // NW
CHUNK = 2000
NCHUNK = EPW // CHUNK
GRP = CHUNK // 16
UNROLL = 5
NPAD = 100096
RPT = NPAD // NS

_PB = 4000
_PG = N_NODES // _PB


def _prep_body(ucw_ref, up_ref, uo_ref, t_ref, t2_ref):
    x = up_ref[...]
    m = jnp.max(x, axis=1, keepdims=True)
    e = jnp.exp(x - m)
    sm = e / jnp.sum(e, axis=1, keepdims=True)
    u = x + sm * ucw_ref[0]
    uo_ref[...] = u
    col = lax.broadcasted_iota(jnp.int32, u.shape, 1)
    valid = col < NF
    uc = jnp.where(valid, u, 0.0)
    t_ref[...] = jnp.where(valid, jnp.exp(-uc), 0.0)
    t2_ref[...] = jnp.where(valid, jnp.exp(uc), 0.0)


_prep = pl.pallas_call(
    _prep_body,
    grid=(_PG,),
    in_specs=[
        pl.BlockSpec(memory_space=pltpu.SMEM),
        pl.BlockSpec((_PB, 8), lambda i: (i, 0)),
    ],
    out_specs=[
        pl.BlockSpec((_PB, 8), lambda i: (i, 0)),
        pl.BlockSpec((_PB, 8), lambda i: (i, 0)),
        pl.BlockSpec((_PB, 8), lambda i: (i, 0)),
    ],
    out_shape=[
        jax.ShapeDtypeStruct((N_NODES, 8), jnp.float32),
        jax.ShapeDtypeStruct((N_NODES, 8), jnp.float32),
        jax.ShapeDtypeStruct((N_NODES, 8), jnp.float32),
    ],
)


def _combine_body(u_ref, a_ref, b_ref, o_ref):
    o_ref[...] = u_ref[...] + a_ref[...] + b_ref[...]


_combine = pl.pallas_call(
    _combine_body,
    grid=(_PG,),
    in_specs=[pl.BlockSpec((_PB, 8), lambda i: (i, 0))] * 3,
    out_specs=pl.BlockSpec((_PB, 8), lambda i: (i, 0)),
    out_shape=jax.ShapeDtypeStruct((N_NODES, 8), jnp.float32),
)


def _edge_body(t1_hbm, t2_hbm, bin_hbm, i1_hbm, i2_hbm, w_hbm, z_hbm,
               bp_hbm, acc_hbm,
               idx1_v, idx2_v, bin_v, g1_v, g2_v, s1_v, s2_v, bp_v, w_v,
               acc_sh, sem1, sem2, sem_lin, sem_sc):
    c = lax.axis_index("c")
    s = lax.axis_index("s")
    wid = s * NC + c

    row0 = pl.multiple_of(s * RPT, 8)
    pltpu.sync_copy(z_hbm.at[pl.ds(row0, RPT)], acc_sh.at[pl.ds(row0, RPT)])
    pltpu.sync_copy(z_hbm.at[pl.ds(0, CHUNK)], s1_v)
    pltpu.sync_copy(z_hbm.at[pl.ds(0, CHUNK)], s2_v)
    pltpu.sync_copy(w_hbm, w_v)
    plsc.subcore_barrier()

    def chunk_body(k, carry):
        lin = [pltpu.async_copy(i1_hbm.at[wid, k], idx1_v, sem_lin),
               pltpu.async_copy(i2_hbm.at[wid, k], idx2_v, sem_lin),
               pltpu.async_copy(bin_hbm.at[wid, k], bin_v, sem_lin)]
        for d in lin:
            d.wait()

        d1 = pltpu.async_copy(t1_hbm.at[idx1_v], g1_v, sem1)
        d2 = pltpu.async_copy(t2_hbm.at[idx2_v], g2_v, sem2)
        d1.wait()
        d2.wait()

        def grp_body(g0, carry2):
            for u in range(UNROLL):
                g = g0 * UNROLL + u
                rows = g * 16 + lax.iota(jnp.int32, 16)
                b = bin_v[pl.ds(g * 16, 16)]
                cexp = jnp.exp(-b)
                ssum = jnp.zeros((16,), jnp.float32)
                for i in range(NF):
                    ci = jnp.full((16,), i, jnp.int32)
                    a = plsc.load_gather(g1_v, [rows, ci])
                    dp = plsc.load_gather(g2_v, [rows, ci])
                    den = a + cexp + dp
                    y = plsc.bitcast(
                        jnp.int32(0x7EF311C3) - plsc.bitcast(den, jnp.int32),
                        jnp.float32)
                    y = y * (2.0 - den * y)
                    y = y * (2.0 - den * y)
                    y = y * (2.0 - den * y)
                    r = w_v[i] * y
                    plsc.store_scatter(s1_v, [rows, ci], -a * r)
                    plsc.store_scatter(s2_v, [rows, ci], dp * r)
                    ssum = ssum + r
                bp_v[pl.ds(g * 16, 16)] = b - cexp * ssum
            return carry2

        lax.fori_loop(0, GRP // UNROLL, grp_body, 0)

        pltpu.sync_copy(s1_v, acc_sh.at[idx1_v], add=True)
        pltpu.sync_copy(s2_v, acc_sh.at[idx2_v], add=True)
        pltpu.sync_copy(bp_v, bp_hbm.at[wid, k])
        return carry

    lax.fori_loop(0, NCHUNK, chunk_body, 0)

    plsc.subcore_barrier()
    pltpu.sync_copy(acc_sh.at[pl.ds(row0, RPT)],
                    acc_hbm.at[c].at[pl.ds(row0, RPT)])


_edge = functools.partial(
    pl.kernel,
    out_type=[
        jax.ShapeDtypeStruct((NW, NCHUNK, CHUNK), jnp.float32),
        jax.ShapeDtypeStruct((NC, NPAD, 8), jnp.float32),
    ],
    mesh=plsc.VectorSubcoreMesh(core_axis_name="c", subcore_axis_name="s",
                                num_cores=NC, num_subcores=NS),
    compiler_params=pltpu.CompilerParams(needs_layout_passes=False,
                                         use_tc_tiling_on_sc=False),
    scratch_types=[
        pltpu.VMEM((CHUNK,), jnp.int32),
        pltpu.VMEM((CHUNK,), jnp.int32),
        pltpu.VMEM((CHUNK,), jnp.float32),
        pltpu.VMEM((CHUNK, 8), jnp.float32),
        pltpu.VMEM((CHUNK, 8), jnp.float32),
        pltpu.VMEM((CHUNK, 8), jnp.float32),
        pltpu.VMEM((CHUNK, 8), jnp.float32),
        pltpu.VMEM((CHUNK,), jnp.float32),
        pltpu.VMEM((8, 16), jnp.float32),
        pltpu.VMEM_SHARED((NPAD, 8), jnp.float32),
        pltpu.SemaphoreType.DMA,
        pltpu.SemaphoreType.DMA,
        pltpu.SemaphoreType.DMA,
        pltpu.SemaphoreType.DMA,
    ],
)(_edge_body)


@jax.jit
def kernel(unary, binary, index1, index2, unary_cw, binary_cw):
    pad = jnp.full((N_NODES, 2), -1e30, jnp.float32)
    unary_p = jnp.concatenate([unary, pad], axis=1)
    u_p, t1, t2 = _prep(unary_cw, unary_p)

    w_t = jnp.zeros((8, 16), jnp.float32).at[:NF].set(
        jnp.broadcast_to(binary_cw[:, None], (NF, 16)))
    zeros = jnp.zeros((NPAD, 8), jnp.float32)
    i1r = index1.reshape(NW, NCHUNK, CHUNK)
    i2r = index2.reshape(NW, NCHUNK, CHUNK)
    binr = binary.reshape(NW, NCHUNK, CHUNK)

    bp_r, acc = _edge(t1, t2, binr, i1r, i2r, w_t, zeros)

    up_p = _combine(u_p, acc[0, :N_NODES], acc[1, :N_NODES])
    return up_p[:, :NF], bp_r.reshape(N_EDGES, 1)


# speedup vs baseline: 1.1299x; 1.1299x over previous
import functools

import jax
import jax.numpy as jnp
from jax import lax
from jax.experimental import pallas as pl
from jax.experimental.pallas import tpu as pltpu
from jax.experimental.pallas import tpu_sc as plsc

N_NODES = 100000
N_EDGES = 3200000
NF = 6
NC, NS = 2, 16
NW = NC * NS
EPW = N_EDGES // NW
CHUNK = 2000
NCHUNK = EPW // CHUNK
GRP = CHUNK // 16
UNROLL = 5
NPAD = 100096
RPT = NPAD // NS

_PB = 4000
_PG = N_NODES // _PB


def _prep_body(ucw_ref, up_ref, uo_ref, t_ref, t2_ref):
    x = up_ref[...]
    m = jnp.max(x, axis=1, keepdims=True)
    e = jnp.exp(x - m)
    sm = e / jnp.sum(e, axis=1, keepdims=True)
    u = x + sm * ucw_ref[0]
    uo_ref[...] = u
    col = lax.broadcasted_iota(jnp.int32, u.shape, 1)
    valid = col < NF
    uc = jnp.where(valid, u, 0.0)
    t_ref[...] = jnp.where(valid, jnp.exp(-uc), 0.0)
    t2_ref[...] = jnp.where(valid, jnp.exp(uc), 0.0)


_prep = pl.pallas_call(
    _prep_body,
    grid=(_PG,),
    in_specs=[
        pl.BlockSpec(memory_space=pltpu.SMEM),
        pl.BlockSpec((_PB, 8), lambda i: (i, 0)),
    ],
    out_specs=[
        pl.BlockSpec((_PB, 8), lambda i: (i, 0)),
        pl.BlockSpec((_PB, 8), lambda i: (i, 0)),
        pl.BlockSpec((_PB, 8), lambda i: (i, 0)),
    ],
    out_shape=[
        jax.ShapeDtypeStruct((N_NODES, 8), jnp.float32),
        jax.ShapeDtypeStruct((N_NODES, 8), jnp.float32),
        jax.ShapeDtypeStruct((N_NODES, 8), jnp.float32),
    ],
)


def _combine_body(u_ref, a_ref, b_ref, o_ref):
    o_ref[...] = u_ref[...] + a_ref[...] + b_ref[...]


_combine = pl.pallas_call(
    _combine_body,
    grid=(_PG,),
    in_specs=[pl.BlockSpec((_PB, 8), lambda i: (i, 0))] * 3,
    out_specs=pl.BlockSpec((_PB, 8), lambda i: (i, 0)),
    out_shape=jax.ShapeDtypeStruct((N_NODES, 8), jnp.float32),
)


def _edge_body(t1_hbm, t2_hbm, bin_hbm, i1_hbm, i2_hbm, w_hbm, z_hbm,
               bp_hbm, acc_hbm,
               idx1_v, idx2_v, bin_v, g1_v, g2_v, s1_v, s2_v, bp_v, w_v,
               acc_sh, sem1, sem2, sem_lin, sem_sc):
    c = lax.axis_index("c")
    s = lax.axis_index("s")
    wid = s * NC + c

    row0 = pl.multiple_of(s * RPT, 8)
    pltpu.sync_copy(z_hbm.at[pl.ds(row0, RPT)], acc_sh.at[pl.ds(row0, RPT)])
    pltpu.sync_copy(z_hbm.at[pl.ds(0, CHUNK)], s1_v)
    pltpu.sync_copy(z_hbm.at[pl.ds(0, CHUNK)], s2_v)
    pltpu.sync_copy(w_hbm, w_v)
    plsc.subcore_barrier()

    def chunk_body(k, carry):
        lin = [pltpu.async_copy(i1_hbm.at[wid, k], idx1_v, sem_lin),
               pltpu.async_copy(i2_hbm.at[wid, k], idx2_v, sem_lin),
               pltpu.async_copy(bin_hbm.at[wid, k], bin_v, sem_lin)]
        for d in lin:
            d.wait()

        d1 = pltpu.async_copy(t1_hbm.at[idx1_v], g1_v, sem1)
        d2 = pltpu.async_copy(t2_hbm.at[idx2_v], g2_v, sem2)
        d1.wait()
        d2.wait()

        def grp_body(g0, carry2):
            for u in range(UNROLL):
                g = g0 * UNROLL + u
                rows = g * 16 + lax.iota(jnp.int32, 16)
                b = bin_v[pl.ds(g * 16, 16)]
                cexp = jnp.exp(-b)
                ssum = jnp.zeros((16,), jnp.float32)
                for i in range(NF):
                    ci = jnp.full((16,), i, jnp.int32)
                    a = plsc.load_gather(g1_v, [rows, ci])
                    dp = plsc.load_gather(g2_v, [rows, ci])
                    r = w_v[i] / (a + cexp + dp)
                    plsc.store_scatter(s1_v, [rows, ci], -a * r)
                    plsc.store_scatter(s2_v, [rows, ci], dp * r)
                    ssum = ssum + r
                bp_v[pl.ds(g * 16, 16)] = b - cexp * ssum
            return carry2

        lax.fori_loop(0, GRP // UNROLL, grp_body, 0)

        pltpu.sync_copy(s1_v, acc_sh.at[idx1_v], add=True)
        pltpu.sync_copy(s2_v, acc_sh.at[idx2_v], add=True)
        pltpu.sync_copy(bp_v, bp_hbm.at[wid, k])
        return carry

    lax.fori_loop(0, NCHUNK, chunk_body, 0)

    plsc.subcore_barrier()
    pltpu.sync_copy(acc_sh.at[pl.ds(row0, RPT)],
                    acc_hbm.at[c].at[pl.ds(row0, RPT)])


_edge = functools.partial(
    pl.kernel,
    out_type=[
        jax.ShapeDtypeStruct((NW, NCHUNK, CHUNK), jnp.float32),
        jax.ShapeDtypeStruct((NC, NPAD, 8), jnp.float32),
    ],
    mesh=plsc.VectorSubcoreMesh(core_axis_name="c", subcore_axis_name="s",
                                num_cores=NC, num_subcores=NS),
    compiler_params=pltpu.CompilerParams(needs_layout_passes=False,
                                         use_tc_tiling_on_sc=False),
    scratch_types=[
        pltpu.VMEM((CHUNK,), jnp.int32),
        pltpu.VMEM((CHUNK,), jnp.int32),
        pltpu.VMEM((CHUNK,), jnp.float32),
        pltpu.VMEM((CHUNK, 8), jnp.float32),
        pltpu.VMEM((CHUNK, 8), jnp.float32),
        pltpu.VMEM((CHUNK, 8), jnp.float32),
        pltpu.VMEM((CHUNK, 8), jnp.float32),
        pltpu.VMEM((CHUNK,), jnp.float32),
        pltpu.VMEM((8, 16), jnp.float32),
        pltpu.VMEM_SHARED((NPAD, 8), jnp.float32),
        pltpu.SemaphoreType.DMA,
        pltpu.SemaphoreType.DMA,
        pltpu.SemaphoreType.DMA,
        pltpu.SemaphoreType.DMA,
    ],
)(_edge_body)


@jax.jit
def kernel(unary, binary, index1, index2, unary_cw, binary_cw):
    pad = jnp.full((N_NODES, 2), -1e30, jnp.float32)
    unary_p = jnp.concatenate([unary, pad], axis=1)
    u_p, t1, t2 = _prep(unary_cw, unary_p)

    w_t = jnp.zeros((8, 16), jnp.float32).at[:NF].set(
        jnp.broadcast_to(binary_cw[:, None], (NF, 16)))
    zeros = jnp.zeros((NPAD, 8), jnp.float32)
    i1r = index1.reshape(NW, NCHUNK, CHUNK)
    i2r = index2.reshape(NW, NCHUNK, CHUNK)
    binr = binary.reshape(NW, NCHUNK, CHUNK)

    bp_r, acc = _edge(t1, t2, binr, i1r, i2r, w_t, zeros)

    up_p = _combine(u_p, acc[0, :N_NODES], acc[1, :N_NODES])
    return up_p[:, :NF], bp_r.reshape(N_EDGES, 1)


# speedup vs baseline: 1.1387x; 1.0078x over previous
import functools

import jax
import jax.numpy as jnp
from jax import lax
from jax.experimental import pallas as pl
from jax.experimental.pallas import tpu as pltpu
from jax.experimental.pallas import tpu_sc as plsc

N_NODES = 100000
N_EDGES = 3200000
NF = 6
NC, NS = 2, 16
NW = NC * NS
EPW = N_EDGES // NW
CHUNK = 2000
NCHUNK = EPW // CHUNK
GRP = CHUNK // 16
UNROLL = 5
NPAD = 100096
RPT = NPAD // NS

_PB = 4000
_PG = N_NODES // _PB


def _prep_body(ucw_ref, up_ref, uo_ref, t_ref, t2_ref):
    x = up_ref[...]
    m = jnp.max(x, axis=1, keepdims=True)
    e = jnp.exp(x - m)
    sm = e / jnp.sum(e, axis=1, keepdims=True)
    u = x + sm * ucw_ref[0]
    uo_ref[...] = u
    col = lax.broadcasted_iota(jnp.int32, u.shape, 1)
    valid = col < NF
    uc = jnp.where(valid, u, 0.0)
    t_ref[...] = jnp.where(valid, jnp.exp(-uc), 0.0)
    t2_ref[...] = jnp.where(valid, jnp.exp(uc), 0.0)


_prep = pl.pallas_call(
    _prep_body,
    grid=(_PG,),
    in_specs=[
        pl.BlockSpec(memory_space=pltpu.SMEM),
        pl.BlockSpec((_PB, 8), lambda i: (i, 0)),
    ],
    out_specs=[
        pl.BlockSpec((_PB, 8), lambda i: (i, 0)),
        pl.BlockSpec((_PB, 8), lambda i: (i, 0)),
        pl.BlockSpec((_PB, 8), lambda i: (i, 0)),
    ],
    out_shape=[
        jax.ShapeDtypeStruct((N_NODES, 8), jnp.float32),
        jax.ShapeDtypeStruct((N_NODES, 8), jnp.float32),
        jax.ShapeDtypeStruct((N_NODES, 8), jnp.float32),
    ],
)


def _combine_body(u_ref, a_ref, b_ref, o_ref):
    o_ref[...] = u_ref[...] + a_ref[...] + b_ref[...]


_combine = pl.pallas_call(
    _combine_body,
    grid=(_PG,),
    in_specs=[pl.BlockSpec((_PB, 8), lambda i: (i, 0))] * 3,
    out_specs=pl.BlockSpec((_PB, 8), lambda i: (i, 0)),
    out_shape=jax.ShapeDtypeStruct((N_NODES, 8), jnp.float32),
)


def _edge_body(t1_hbm, t2_hbm, bin_hbm, i1_hbm, i2_hbm, w_hbm, z_hbm,
               bp_hbm, acc_hbm,
               idx1_v, idx2_v, bin_v, g1_v, g2_v, s1_v, s2_v, bp_v, w_v,
               acc_sh, sem1, sem2, sem_lin, sem_sc):
    c = lax.axis_index("c")
    s = lax.axis_index("s")
    wid = s * NC + c

    row0 = pl.multiple_of(s * RPT, 8)
    pltpu.sync_copy(z_hbm.at[pl.ds(row0, RPT)], acc_sh.at[pl.ds(row0, RPT)])
    pltpu.sync_copy(z_hbm.at[pl.ds(0, CHUNK)], s1_v)
    pltpu.sync_copy(z_hbm.at[pl.ds(0, CHUNK)], s2_v)
    pltpu.sync_copy(w_hbm, w_v)
    plsc.subcore_barrier()

    def chunk_body(k, carry):
        off = pl.multiple_of(wid * EPW + k * CHUNK, 8)
        lin = [pltpu.async_copy(i1_hbm.at[pl.ds(off, CHUNK)], idx1_v, sem_lin),
               pltpu.async_copy(i2_hbm.at[pl.ds(off, CHUNK)], idx2_v, sem_lin),
               pltpu.async_copy(bin_hbm.at[pl.ds(off, CHUNK)], bin_v, sem_lin)]
        for d in lin:
            d.wait()

        d1 = pltpu.async_copy(t1_hbm.at[idx1_v], g1_v, sem1)
        d2 = pltpu.async_copy(t2_hbm.at[idx2_v], g2_v, sem2)
        d1.wait()
        d2.wait()

        def grp_body(g, carry2):
            rows = g * 16 + lax.iota(jnp.int32, 16)
            b = bin_v[pl.ds(g * 16, 16)]
            cexp = jnp.exp(-b)
            ssum = jnp.zeros((16,), jnp.float32)
            for i in range(NF):
                ci = jnp.full((16,), i, jnp.int32)
                a = plsc.load_gather(g1_v, [rows, ci])
                dp = plsc.load_gather(g2_v, [rows, ci])
                r = w_v[i] / (a + cexp + dp)
                plsc.store_scatter(s1_v, [rows, ci], -a * r)
                plsc.store_scatter(s2_v, [rows, ci], dp * r)
                ssum = ssum + r
            bp_v[pl.ds(g * 16, 16)] = b - cexp * ssum
            return carry2

        lax.fori_loop(0, GRP, grp_body, 0)

        pltpu.sync_copy(s1_v, acc_sh.at[idx1_v], add=True)
        pltpu.sync_copy(s2_v, acc_sh.at[idx2_v], add=True)
        pltpu.sync_copy(bp_v, bp_hbm.at[pl.ds(off, CHUNK)])
        return carry

    lax.fori_loop(0, NCHUNK, chunk_body, 0)

    plsc.subcore_barrier()
    pltpu.sync_copy(acc_sh.at[pl.ds(row0, RPT)],
                    acc_hbm.at[c].at[pl.ds(row0, RPT)])


_edge = functools.partial(
    pl.kernel,
    out_type=[
        jax.ShapeDtypeStruct((N_EDGES,), jnp.float32),
        jax.ShapeDtypeStruct((NC, NPAD, 8), jnp.float32),
    ],
    mesh=plsc.VectorSubcoreMesh(core_axis_name="c", subcore_axis_name="s",
                                num_cores=NC, num_subcores=NS),
    compiler_params=pltpu.CompilerParams(needs_layout_passes=False,
                                         use_tc_tiling_on_sc=False),
    scratch_types=[
        pltpu.VMEM((CHUNK,), jnp.int32),
        pltpu.VMEM((CHUNK,), jnp.int32),
        pltpu.VMEM((CHUNK,), jnp.float32),
        pltpu.VMEM((CHUNK, 8), jnp.float32),
        pltpu.VMEM((CHUNK, 8), jnp.float32),
        pltpu.VMEM((CHUNK, 8), jnp.float32),
        pltpu.VMEM((CHUNK, 8), jnp.float32),
        pltpu.VMEM((CHUNK,), jnp.float32),
        pltpu.VMEM((8, 16), jnp.float32),
        pltpu.VMEM_SHARED((NPAD, 8), jnp.float32),
        pltpu.SemaphoreType.DMA,
        pltpu.SemaphoreType.DMA,
        pltpu.SemaphoreType.DMA,
        pltpu.SemaphoreType.DMA,
    ],
)(_edge_body)


@jax.jit
def kernel(unary, binary, index1, index2, unary_cw, binary_cw):
    pad = jnp.full((N_NODES, 2), -1e30, jnp.float32)
    unary_p = jnp.concatenate([unary, pad], axis=1)
    u_p, t1, t2 = _prep(unary_cw, unary_p)

    w_t = jnp.zeros((8, 16), jnp.float32).at[:NF].set(
        jnp.broadcast_to(binary_cw[:, None], (NF, 16)))
    zeros = jnp.zeros((NPAD, 8), jnp.float32)
    bp_r, acc = _edge(t1, t2, binary.reshape(N_EDGES), index1, index2,
                      w_t, zeros)

    up_p = _combine(u_p, acc[0, :N_NODES], acc[1, :N_NODES])
    return up_p[:, :NF], bp_r.reshape(N_EDGES, 1)


# speedup vs baseline: 1.2572x; 1.1040x over previous
import functools

import jax
import jax.numpy as jnp
from jax import lax
from jax.experimental import pallas as pl
from jax.experimental.pallas import tpu as pltpu
from jax.experimental.pallas import tpu_sc as plsc

N_NODES = 100000
N_EDGES = 3200000
NF = 6
NC, NS = 2, 16
NW = NC * NS
EPW = N_EDGES // NW
CHUNK = 800
NCHUNK = EPW // CHUNK
GRP = CHUNK // 16
UNROLL = 5
NPAD = 100096
RPT = NPAD // NS

_PB = 4000
_PG = N_NODES // _PB


def _prep_body(ucw_ref, up_ref, uo_ref, t_ref, t2_ref):
    x = up_ref[...]
    m = jnp.max(x, axis=1, keepdims=True)
    e = jnp.exp(x - m)
    sm = e / jnp.sum(e, axis=1, keepdims=True)
    u = x + sm * ucw_ref[0]
    uo_ref[...] = u
    col = lax.broadcasted_iota(jnp.int32, u.shape, 1)
    valid = col < NF
    uc = jnp.where(valid, u, 0.0)
    t_ref[...] = jnp.where(valid, jnp.exp(-uc), 0.0)
    t2_ref[...] = jnp.where(valid, jnp.exp(uc), 0.0)


_prep = pl.pallas_call(
    _prep_body,
    grid=(_PG,),
    in_specs=[
        pl.BlockSpec(memory_space=pltpu.SMEM),
        pl.BlockSpec((_PB, 8), lambda i: (i, 0)),
    ],
    out_specs=[
        pl.BlockSpec((_PB, 8), lambda i: (i, 0)),
        pl.BlockSpec((_PB, 8), lambda i: (i, 0)),
        pl.BlockSpec((_PB, 8), lambda i: (i, 0)),
    ],
    out_shape=[
        jax.ShapeDtypeStruct((N_NODES, 8), jnp.float32),
        jax.ShapeDtypeStruct((N_NODES, 8), jnp.float32),
        jax.ShapeDtypeStruct((N_NODES, 8), jnp.float32),
    ],
)


def _combine_body(u_ref, a_ref, b_ref, o_ref):
    o_ref[...] = u_ref[...] + a_ref[...] + b_ref[...]


_combine = pl.pallas_call(
    _combine_body,
    grid=(_PG,),
    in_specs=[pl.BlockSpec((_PB, 8), lambda i: (i, 0))] * 3,
    out_specs=pl.BlockSpec((_PB, 8), lambda i: (i, 0)),
    out_shape=jax.ShapeDtypeStruct((N_NODES, 8), jnp.float32),
)


def _edge_body(t1_hbm, t2_hbm, bin_hbm, i1_hbm, i2_hbm, w_hbm, z_hbm,
               bp_hbm, acc_hbm,
               idx1_v, idx2_v, bin_v, g1_v, g2_v, s1_v, s2_v, bp_v, w_v,
               acc_sh, sem1a, sem1b, sem2a, sem2b, sem_lin):
    c = lax.axis_index("c")
    s = lax.axis_index("s")
    wid = s * NC + c
    gsems = ((sem1a, sem2a), (sem1b, sem2b))

    row0 = pl.multiple_of(s * RPT, 8)
    pltpu.sync_copy(z_hbm.at[pl.ds(row0, RPT)], acc_sh.at[pl.ds(row0, RPT)])
    pltpu.sync_copy(z_hbm.at[pl.ds(0, CHUNK)], s1_v)
    pltpu.sync_copy(z_hbm.at[pl.ds(0, CHUNK)], s2_v)
    pltpu.sync_copy(w_hbm, w_v)
    plsc.subcore_barrier()

    def lin_load(k, p):
        off = pl.multiple_of(wid * EPW + k * CHUNK, 8)
        lin = [pltpu.async_copy(i1_hbm.at[pl.ds(off, CHUNK)], idx1_v.at[p], sem_lin),
               pltpu.async_copy(i2_hbm.at[pl.ds(off, CHUNK)], idx2_v.at[p], sem_lin),
               pltpu.async_copy(bin_hbm.at[pl.ds(off, CHUNK)], bin_v.at[p], sem_lin)]
        for d in lin:
            d.wait()

    def fire_gathers(p):
        pltpu.async_copy(t1_hbm.at[idx1_v.at[p]], g1_v.at[p], gsems[p][0])
        pltpu.async_copy(t2_hbm.at[idx2_v.at[p]], g2_v.at[p], gsems[p][1])

    def wait_gathers(p):
        pltpu.make_async_copy(t1_hbm.at[idx1_v.at[p]], g1_v.at[p],
                              gsems[p][0]).wait()
        pltpu.make_async_copy(t2_hbm.at[idx2_v.at[p]], g2_v.at[p],
                              gsems[p][1]).wait()

    def process(k, p, q):
        kn = jnp.minimum(k + 1, NCHUNK - 1)
        lin_load(kn, q)
        fire_gathers(q)
        wait_gathers(p)

        def grp_body(g, carry2):
            rows = g * 16 + lax.iota(jnp.int32, 16)
            b = bin_v[p, pl.ds(g * 16, 16)]
            cexp = jnp.exp(-b)
            ssum = jnp.zeros((16,), jnp.float32)
            for i in range(NF):
                ci = jnp.full((16,), i, jnp.int32)
                a = plsc.load_gather(g1_v.at[p], [rows, ci])
                dp = plsc.load_gather(g2_v.at[p], [rows, ci])
                r = w_v[i] / (a + cexp + dp)
                plsc.store_scatter(s1_v, [rows, ci], -a * r)
                plsc.store_scatter(s2_v, [rows, ci], dp * r)
                ssum = ssum + r
            bp_v[pl.ds(g * 16, 16)] = b - cexp * ssum
            return carry2

        lax.fori_loop(0, GRP, grp_body, 0)

        off = pl.multiple_of(wid * EPW + k * CHUNK, 8)
        pltpu.sync_copy(s1_v, acc_sh.at[idx1_v.at[p]], add=True)
        pltpu.sync_copy(s2_v, acc_sh.at[idx2_v.at[p]], add=True)
        pltpu.sync_copy(bp_v, bp_hbm.at[pl.ds(off, CHUNK)])

    lin_load(0, 0)
    fire_gathers(0)
    process(0, 0, 1)

    def pair_body(t, carry):
        process(1 + 2 * t, 1, 0)
        process(2 + 2 * t, 0, 1)
        return carry

    lax.fori_loop(0, (NCHUNK - 1) // 2, pair_body, 0)
    wait_gathers(1)

    plsc.subcore_barrier()
    pltpu.sync_copy(acc_sh.at[pl.ds(row0, RPT)],
                    acc_hbm.at[c].at[pl.ds(row0, RPT)])


_edge = functools.partial(
    pl.kernel,
    out_type=[
        jax.ShapeDtypeStruct((N_EDGES,), jnp.float32),
        jax.ShapeDtypeStruct((NC, NPAD, 8), jnp.float32),
    ],
    mesh=plsc.VectorSubcoreMesh(core_axis_name="c", subcore_axis_name="s",
                                num_cores=NC, num_subcores=NS),
    compiler_params=pltpu.CompilerParams(needs_layout_passes=False,
                                         use_tc_tiling_on_sc=False),
    scratch_types=[
        pltpu.VMEM((2, CHUNK), jnp.int32),
        pltpu.VMEM((2, CHUNK), jnp.int32),
        pltpu.VMEM((2, CHUNK), jnp.float32),
        pltpu.VMEM((2, CHUNK, 8), jnp.float32),
        pltpu.VMEM((2, CHUNK, 8), jnp.float32),
        pltpu.VMEM((CHUNK, 8), jnp.float32),
        pltpu.VMEM((CHUNK, 8), jnp.float32),
        pltpu.VMEM((CHUNK,), jnp.float32),
        pltpu.VMEM((8, 16), jnp.float32),
        pltpu.VMEM_SHARED((NPAD, 8), jnp.float32),
        pltpu.SemaphoreType.DMA,
        pltpu.SemaphoreType.DMA,
        pltpu.SemaphoreType.DMA,
        pltpu.SemaphoreType.DMA,
        pltpu.SemaphoreType.DMA,
    ],
)(_edge_body)


@jax.jit
def kernel(unary, binary, index1, index2, unary_cw, binary_cw):
    pad = jnp.full((N_NODES, 2), -1e30, jnp.float32)
    unary_p = jnp.concatenate([unary, pad], axis=1)
    u_p, t1, t2 = _prep(unary_cw, unary_p)

    w_t = jnp.zeros((8, 16), jnp.float32).at[:NF].set(
        jnp.broadcast_to(binary_cw[:, None], (NF, 16)))
    zeros = jnp.zeros((NPAD, 8), jnp.float32)
    bp_r, acc = _edge(t1, t2, binary.reshape(N_EDGES), index1, index2,
                      w_t, zeros)

    up_p = _combine(u_p, acc[0, :N_NODES], acc[1, :N_NODES])
    return up_p[:, :NF], bp_r.reshape(N_EDGES, 1)
